# baseline (speedup 1.0000x reference)
import functools
import math

import jax
import jax.numpy as jnp
from jax.experimental import pallas as pl
from jax.experimental.pallas import tpu as pltpu

_BN_EPS = 1e-5
_INV_SQRT2 = 0.7071067811865476


def _round_up(x, m):
    return (x + m - 1) // m * m


def _stats_kernel(x_ref, w1_ref, gamma_ref, beta_ref,
                  w1s_ref, shift2_ref, gram_ref, csum_ref, *,
                  n_steps, true_b):
    i = pl.program_id(0)

    @pl.when(i == 0)
    def _():
        gram_ref[...] = jnp.zeros_like(gram_ref)
        csum_ref[...] = jnp.zeros_like(csum_ref)

    xb16 = x_ref[...].astype(jnp.bfloat16)
    tb = xb16.shape[0]
    gram_ref[...] += jax.lax.dot_general(
        xb16, xb16,
        dimension_numbers=(((0,), (0,)), ((), ())),
        preferred_element_type=jnp.float32)
    ones = jnp.ones((8, tb), jnp.bfloat16)
    csum_ref[...] += jnp.dot(ones, xb16, preferred_element_type=jnp.float32)

    @pl.when(i == n_steps - 1)
    def _():
        inv_b = 1.0 / true_b
        w1 = w1_ref[...]
        cs = jnp.sum(csum_ref[...], axis=0, keepdims=True) * 0.125
        mean = jnp.dot(cs, w1, preferred_element_type=jnp.float32) * inv_b
        m = jnp.dot(gram_ref[...], w1, preferred_element_type=jnp.float32)
        ex2 = jnp.sum(w1 * m, axis=0, keepdims=True) * inv_b
        var = jnp.maximum(ex2 - mean * mean, 0.0)
        scale = gamma_ref[...] * jax.lax.rsqrt(var + _BN_EPS)
        sc = scale * _INV_SQRT2
        w1s_ref[...] = (w1 * sc).astype(jnp.bfloat16)
        shift2_ref[...] = (beta_ref[...] - mean * scale) * _INV_SQRT2


def _apply_kernel(x_ref, w1s_ref, shift2_ref, w2s_ref, b2_ref, o_ref, *,
                  chunk_rows):
    w1s = w1s_ref[...]
    w2s = w2s_ref[...]
    shift2 = shift2_ref[...]
    b2v = b2_ref[...]
    tb = x_ref.shape[0]
    for r0 in range(0, tb, chunk_rows):
        rows = pl.ds(r0, chunk_rows)
        u = jnp.dot(x_ref[rows, :].astype(jnp.bfloat16), w1s,
                    preferred_element_type=jnp.float32) + shift2
        g = u + u * jax.lax.erf(u)
        out = jnp.dot(g.astype(jnp.bfloat16), w2s,
                      preferred_element_type=jnp.float32) + b2v
        o_ref[rows, :] = out.astype(o_ref.dtype)


def kernel(z, w1, b1, gamma, beta, w2, b2, *,
           stats_tile_b=2048, tile_b=1024, chunk_rows=256):
    del b1
    B, in_dim = z.shape
    H = w1.shape[1]
    out_dim = w2.shape[1]
    f32 = jnp.float32

    tb1 = min(stats_tile_b, max(8, _round_up(B, 8)))
    tb2 = min(tile_b, max(8, _round_up(B, 8)))
    b_p = _round_up(B, tb1 * tb2 // math.gcd(tb1, tb2))
    z = z.astype(f32)
    if b_p != B:
        z = jnp.pad(z, ((0, b_p - B), (0, 0)))
    n1 = b_p // tb1
    n2 = b_p // tb2

    w1f = w1.astype(f32)
    w2s = (w2.astype(f32) * _INV_SQRT2).astype(jnp.bfloat16)
    gamma = gamma.astype(f32).reshape(1, H)
    beta = beta.astype(f32).reshape(1, H)
    b2 = b2.astype(f32).reshape(1, out_dim)

    const1 = lambda i: (0, 0)
    w1s, shift2 = pl.pallas_call(
        functools.partial(_stats_kernel, n_steps=n1, true_b=B),
        out_shape=(jax.ShapeDtypeStruct((in_dim, H), jnp.bfloat16),
                   jax.ShapeDtypeStruct((1, H), f32)),
        grid=(n1,),
        in_specs=[pl.BlockSpec((tb1, in_dim), lambda i: (i, 0)),
                  pl.BlockSpec((in_dim, H), const1),
                  pl.BlockSpec((1, H), const1),
                  pl.BlockSpec((1, H), const1)],
        out_specs=(pl.BlockSpec((in_dim, H), const1),
                   pl.BlockSpec((1, H), const1)),
        scratch_shapes=[pltpu.VMEM((in_dim, in_dim), f32),
                        pltpu.VMEM((8, in_dim), f32)],
        compiler_params=pltpu.CompilerParams(
            dimension_semantics=("arbitrary",)),
    )(z, w1f, gamma, beta)

    cr = math.gcd(chunk_rows, tb2)
    const2 = lambda i: (0, 0)
    out = pl.pallas_call(
        functools.partial(_apply_kernel, chunk_rows=cr),
        out_shape=jax.ShapeDtypeStruct((b_p, out_dim), f32),
        grid=(n2,),
        in_specs=[pl.BlockSpec((tb2, in_dim), lambda i: (i, 0)),
                  pl.BlockSpec((in_dim, H), const2),
                  pl.BlockSpec((1, H), const2),
                  pl.BlockSpec((H, out_dim), const2),
                  pl.BlockSpec((1, out_dim), const2)],
        out_specs=pl.BlockSpec((tb2, out_dim), lambda i: (i, 0)),
        compiler_params=pltpu.CompilerParams(
            dimension_semantics=("arbitrary",)),
    )(z, w1s, shift2, w2s, b2)

    if b_p != B:
        out = out[:B]
    return out


# speedup vs baseline: 1.1047x; 1.1047x over previous
import functools
import math

import jax
import jax.numpy as jnp
from jax.experimental import pallas as pl
from jax.experimental.pallas import tpu as pltpu

_BN_EPS = 1e-5
_INV_SQRT2 = 0.7071067811865476


def _round_up(x, m):
    return (x + m - 1) // m * m


def _fused_kernel(x_ref, w1_ref, gamma_ref, beta_ref, w2_ref, b2_ref,
                  o_ref, gram_ref, csum_ref, w1s_ref, shift2_ref, w2s_ref,
                  *, n_steps, true_b, sub_rows):
    i = pl.program_id(0)

    @pl.when(i == 0)
    def _():
        gram_ref[...] = jnp.zeros_like(gram_ref)
        csum_ref[...] = jnp.zeros_like(csum_ref)

    @pl.when(i < n_steps)
    def _():
        xb16 = x_ref[...].astype(jnp.bfloat16)
        tb = xb16.shape[0]
        gram_ref[...] += jax.lax.dot_general(
            xb16, xb16,
            dimension_numbers=(((0,), (0,)), ((), ())),
            preferred_element_type=jnp.float32)
        ones = jnp.ones((8, tb), jnp.bfloat16)
        csum_ref[...] += jnp.dot(ones, xb16, preferred_element_type=jnp.float32)

    @pl.when(i == n_steps - 1)
    def _():
        inv_b = 1.0 / true_b
        w1 = w1_ref[...]
        cs = jnp.sum(csum_ref[...], axis=0, keepdims=True) * 0.125
        mean = jnp.dot(cs, w1, preferred_element_type=jnp.float32) * inv_b
        m = jnp.dot(gram_ref[...], w1, preferred_element_type=jnp.float32)
        ex2 = jnp.sum(w1 * m, axis=0, keepdims=True) * inv_b
        var = jnp.maximum(ex2 - mean * mean, 0.0)
        scale = gamma_ref[...] * jax.lax.rsqrt(var + _BN_EPS)
        w1s_ref[...] = (w1 * (scale * _INV_SQRT2)).astype(jnp.bfloat16)
        shift2 = (beta_ref[...] - mean * scale) * _INV_SQRT2
        shift2_ref[...] = jnp.broadcast_to(shift2, shift2_ref.shape)
        w2s_ref[...] = (w2_ref[...] * _INV_SQRT2).astype(jnp.bfloat16)

    @pl.when(i >= n_steps)
    def _():
        w1s = w1s_ref[...]
        w2s = w2s_ref[...]
        shift2 = shift2_ref[0:1, :]
        b2v = b2_ref[...]
        tb = x_ref.shape[0]
        for r0 in range(0, tb, sub_rows):
            rows = pl.ds(r0, sub_rows)
            u = jnp.dot(x_ref[rows, :].astype(jnp.bfloat16), w1s,
                        preferred_element_type=jnp.float32) + shift2
            g = u + u * jax.lax.erf(u)
            out = jnp.dot(g.astype(jnp.bfloat16), w2s,
                          preferred_element_type=jnp.float32) + b2v
            o_ref[rows, :] = out.astype(o_ref.dtype)


def kernel(z, w1, b1, gamma, beta, w2, b2, *, tile_b=2048, sub_rows=1024):
    del b1
    B, in_dim = z.shape
    H = w1.shape[1]
    out_dim = w2.shape[1]
    f32 = jnp.float32

    tb = min(tile_b, max(8, _round_up(B, 8)))
    b_p = _round_up(B, tb)
    z = z.astype(f32)
    if b_p != B:
        z = jnp.pad(z, ((0, b_p - B), (0, 0)))
    n = b_p // tb
    sr = math.gcd(sub_rows, tb)

    w1f = w1.astype(f32)
    w2f = w2.astype(f32)
    gamma = gamma.astype(f32).reshape(1, H)
    beta = beta.astype(f32).reshape(1, H)
    b2 = b2.astype(f32).reshape(1, out_dim)

    x_idx = lambda i: (jax.lax.select(i < n, i, i - n), 0)
    o_idx = lambda i: (jax.lax.select(i < n, 0, i - n), 0)
    const = lambda i: (0, 0)

    out = pl.pallas_call(
        functools.partial(_fused_kernel, n_steps=n, true_b=B, sub_rows=sr),
        out_shape=jax.ShapeDtypeStruct((b_p, out_dim), f32),
        grid=(2 * n,),
        in_specs=[pl.BlockSpec((tb, in_dim), x_idx),
                  pl.BlockSpec((in_dim, H), const),
                  pl.BlockSpec((1, H), const),
                  pl.BlockSpec((1, H), const),
                  pl.BlockSpec((H, out_dim), const),
                  pl.BlockSpec((1, out_dim), const)],
        out_specs=pl.BlockSpec((tb, out_dim), o_idx),
        scratch_shapes=[pltpu.VMEM((in_dim, in_dim), f32),
                        pltpu.VMEM((8, in_dim), f32),
                        pltpu.VMEM((in_dim, H), jnp.bfloat16),
                        pltpu.VMEM((8, H), f32),
                        pltpu.VMEM((H, out_dim), jnp.bfloat16)],
        compiler_params=pltpu.CompilerParams(
            dimension_semantics=("arbitrary",),
            vmem_limit_bytes=60 << 20),
    )(z, w1f, gamma, beta, w2f, b2)

    if b_p != B:
        out = out[:B]
    return out


# speedup vs baseline: 1.1118x; 1.0064x over previous
import functools
import math

import jax
import jax.numpy as jnp
from jax.experimental import pallas as pl
from jax.experimental.pallas import tpu as pltpu

_BN_EPS = 1e-5
_INV_SQRT2 = 0.7071067811865476


def _round_up(x, m):
    return (x + m - 1) // m * m


def _fused_kernel(x_ref, w1_ref, gamma_ref, beta_ref, w2_ref, b2_ref,
                  o_ref, gram_ref, csum_ref, w1s_ref, shift2_ref, w2s_ref,
                  *, n_steps, true_b, sub_rows):
    i = pl.program_id(0)

    @pl.when(i == 0)
    def _():
        gram_ref[...] = jnp.zeros_like(gram_ref)
        csum_ref[...] = jnp.zeros_like(csum_ref)

    @pl.when(i < n_steps)
    def _():
        xb = x_ref[...]
        tb = xb.shape[0]
        gram_ref[...] += jax.lax.dot_general(
            xb, xb,
            dimension_numbers=(((0,), (0,)), ((), ())),
            preferred_element_type=jnp.float32)
        ones = jnp.ones((8, tb), jnp.float32)
        csum_ref[...] += jnp.dot(ones, xb, preferred_element_type=jnp.float32)

    @pl.when(i == n_steps - 1)
    def _():
        inv_b = 1.0 / true_b
        w1 = w1_ref[...]
        cs = jnp.sum(csum_ref[...], axis=0, keepdims=True) * 0.125
        mean = jnp.dot(cs, w1, preferred_element_type=jnp.float32) * inv_b
        m = jnp.dot(gram_ref[...], w1, preferred_element_type=jnp.float32)
        ex2 = jnp.sum(w1 * m, axis=0, keepdims=True) * inv_b
        var = jnp.maximum(ex2 - mean * mean, 0.0)
        scale = gamma_ref[...] * jax.lax.rsqrt(var + _BN_EPS)
        w1s_ref[...] = w1 * (scale * _INV_SQRT2)
        shift2 = (beta_ref[...] - mean * scale) * _INV_SQRT2
        shift2_ref[...] = jnp.broadcast_to(shift2, shift2_ref.shape)
        w2s_ref[...] = w2_ref[...] * _INV_SQRT2

    @pl.when(i >= n_steps)
    def _():
        w1s = w1s_ref[...]
        w2s = w2s_ref[...]
        shift2 = shift2_ref[0:1, :]
        b2v = b2_ref[...]
        tb = x_ref.shape[0]
        for r0 in range(0, tb, sub_rows):
            rows = pl.ds(r0, sub_rows)
            u = jnp.dot(x_ref[rows, :], w1s,
                        preferred_element_type=jnp.float32) + shift2
            g = u + u * jax.lax.erf(u)
            out = jnp.dot(g, w2s,
                          preferred_element_type=jnp.float32) + b2v
            o_ref[rows, :] = out.astype(o_ref.dtype)


def kernel(z, w1, b1, gamma, beta, w2, b2, *, tile_b=2048, sub_rows=1024):
    del b1
    B, in_dim = z.shape
    H = w1.shape[1]
    out_dim = w2.shape[1]
    f32 = jnp.float32

    tb = min(tile_b, max(8, _round_up(B, 8)))
    b_p = _round_up(B, tb)
    z = z.astype(f32)
    if b_p != B:
        z = jnp.pad(z, ((0, b_p - B), (0, 0)))
    n = b_p // tb
    sr = math.gcd(sub_rows, tb)

    w1f = w1.astype(f32)
    w2f = w2.astype(f32)
    gamma = gamma.astype(f32).reshape(1, H)
    beta = beta.astype(f32).reshape(1, H)
    b2 = b2.astype(f32).reshape(1, out_dim)

    x_idx = lambda i: (jax.lax.select(i < n, i, i - n), 0)
    o_idx = lambda i: (jax.lax.select(i < n, 0, i - n), 0)
    const = lambda i: (0, 0)

    out = pl.pallas_call(
        functools.partial(_fused_kernel, n_steps=n, true_b=B, sub_rows=sr),
        out_shape=jax.ShapeDtypeStruct((b_p, out_dim), f32),
        grid=(2 * n,),
        in_specs=[pl.BlockSpec((tb, in_dim), x_idx),
                  pl.BlockSpec((in_dim, H), const),
                  pl.BlockSpec((1, H), const),
                  pl.BlockSpec((1, H), const),
                  pl.BlockSpec((H, out_dim), const),
                  pl.BlockSpec((1, out_dim), const)],
        out_specs=pl.BlockSpec((tb, out_dim), o_idx),
        scratch_shapes=[pltpu.VMEM((in_dim, in_dim), f32),
                        pltpu.VMEM((8, in_dim), f32),
                        pltpu.VMEM((in_dim, H), f32),
                        pltpu.VMEM((8, H), f32),
                        pltpu.VMEM((H, out_dim), f32)],
        compiler_params=pltpu.CompilerParams(
            dimension_semantics=("arbitrary",),
            vmem_limit_bytes=60 << 20),
    )(z, w1f, gamma, beta, w2f, b2)

    if b_p != B:
        out = out[:B]
    return out
